# Initial kernel scaffold; baseline (speedup 1.0000x reference)
import jax
import jax.numpy as jnp
from jax.experimental import pallas as pl


def kernel(x, W1, b1, W2, b2, scale):
    raise NotImplementedError("write your pallas kernel here")


# speedup vs baseline: 34.7583x; 34.7583x over previous
import functools

import jax
import jax.numpy as jnp
from jax.experimental import pallas as pl

IN_DIM = 2048
HID = 1000
HID_PAD = 1024
OUT_DIM = 4096
N_TOK = 8192
TOPK = 409

BLOCK_M = 256


def _gates_kernel(x_ref, w1_ref, b1_ref, w2_ref, b2_ref, scale_ref, o_ref):
    x = x_ref[...]
    h = jnp.dot(x, w1_ref[...], preferred_element_type=jnp.float32)
    h = h + b1_ref[...]
    logits = jnp.dot(h, w2_ref[...], preferred_element_type=jnp.float32)
    logits = logits + b2_ref[...]

    i = jax.lax.bitcast_convert_type(logits, jnp.int32)
    ok = i ^ jax.lax.shift_right_arithmetic(i, 31).__and__(jnp.int32(0x7FFFFFFF))

    int_min = jnp.int32(-2147483648)
    p = jnp.zeros((BLOCK_M, 1), dtype=jnp.int32)
    for bit in range(31, -1, -1):
        if bit == 31:
            c = p | int_min
        else:
            c = p | jnp.int32(1 << bit)
        thr = c ^ int_min
        cnt = jnp.sum((ok >= thr).astype(jnp.int32), axis=1, keepdims=True)
        p = jnp.where(cnt >= TOPK, c, p)
    thr = p ^ int_min
    mask = ok >= thr

    v = scale_ref[...] / (1.0 + jnp.exp(-logits))
    o_ref[...] = jnp.where(mask, v, 0.0)


@jax.jit
def kernel(x, W1, b1, W2, b2, scale):
    W1p = jnp.pad(W1, ((0, 0), (0, HID_PAD - HID)))
    b1p = jnp.pad(b1, (0, HID_PAD - HID)).reshape(1, HID_PAD)
    W2p = jnp.pad(W2, ((0, HID_PAD - HID), (0, 0)))
    b2r = b2.reshape(1, OUT_DIM)
    scaler = scale.reshape(1, OUT_DIM)

    m = x.shape[0]
    grid = (m // BLOCK_M,)
    return pl.pallas_call(
        _gates_kernel,
        grid=grid,
        in_specs=[
            pl.BlockSpec((BLOCK_M, IN_DIM), lambda i: (i, 0)),
            pl.BlockSpec((IN_DIM, HID_PAD), lambda i: (0, 0)),
            pl.BlockSpec((1, HID_PAD), lambda i: (0, 0)),
            pl.BlockSpec((HID_PAD, OUT_DIM), lambda i: (0, 0)),
            pl.BlockSpec((1, OUT_DIM), lambda i: (0, 0)),
            pl.BlockSpec((1, OUT_DIM), lambda i: (0, 0)),
        ],
        out_specs=pl.BlockSpec((BLOCK_M, OUT_DIM), lambda i: (i, 0)),
        out_shape=jax.ShapeDtypeStruct((m, OUT_DIM), jnp.float32),
    )(x, W1p, b1p, W2p, b2r, scaler)
